# baseline (speedup 1.0000x reference)
import functools

import numpy as np
import jax
import jax.numpy as jnp
from jax import lax
from jax.experimental import pallas as pl
from jax.experimental.pallas import tpu as pltpu
from jax.experimental.pallas import tpu_sc as plsc

_NUM_LEN = 13
_N_CAT = 26
_CAT_LEN = 8
_TAU = 0.2
_BATCH = 16384
_TOL = 1e-20
_WIDTH = _NUM_LEN + _N_CAT * _CAT_LEN
_CAT_W = _N_CAT * _CAT_LEN
_LANES = 16


def _rotl(x, r):
    return ((x << np.uint32(r)) | (x >> np.uint32(32 - r))).astype(np.uint32)


def _threefry2x32(k0, k1, c0, c1):
    ks0 = np.uint32(k0)
    ks1 = np.uint32(k1)
    ks2 = np.uint32(ks0 ^ ks1 ^ np.uint32(0x1BD11BDA))
    ks = (ks0, ks1, ks2)
    rot = ((13, 15, 26, 6), (17, 29, 16, 24))
    x0 = (c0 + ks0).astype(np.uint32)
    x1 = (c1 + ks1).astype(np.uint32)
    for i in range(5):
        for r in rot[i % 2]:
            x0 = (x0 + x1).astype(np.uint32)
            x1 = _rotl(x1, r)
            x1 = (x1 ^ x0).astype(np.uint32)
        x0 = (x0 + ks[(i + 1) % 3]).astype(np.uint32)
        x1 = (x1 + ks[(i + 2) % 3] + np.uint32(i + 1)).astype(np.uint32)
    return x0, x1


def _fold_in(k0, k1, data):
    o0, o1 = _threefry2x32(k0, k1, np.zeros(1, np.uint32),
                           np.full(1, data, np.uint32))
    return int(o0[0]), int(o1[0])


def _random_bits(k0, k1, n):
    c0 = np.zeros(n, dtype=np.uint32)
    c1 = np.arange(n, dtype=np.uint32)
    o0, o1 = _threefry2x32(k0, k1, c0, c1)
    return (o0 ^ o1).astype(np.uint32)


def _bits_to_unit_float(bits):
    fb = (bits >> np.uint32(9)) | np.uint32(0x3F800000)
    return fb.view(np.float32) - np.float32(1.0)


@functools.cache
def _noise_table():
    n = _BATCH * _CAT_LEN
    tiny = np.float64(np.finfo(np.float32).tiny)
    cols = []
    for i in range(_N_CAT):
        ka = _fold_in(0, 0, i)
        kb = _fold_in(0, 0, 10000 + i)
        u1 = _bits_to_unit_float(_random_bits(ka[0], ka[1], n)).astype(np.float64)
        g1 = -np.log(-np.log(u1 + _TOL) + _TOL)
        f2 = _bits_to_unit_float(_random_bits(kb[0], kb[1], n)).astype(np.float64)
        u2 = np.maximum(tiny, f2 * (1.0 - tiny) + tiny)
        g2 = -np.log(-np.log(u2))
        cols.append((g1 / _TAU + g2).astype(np.float32).reshape(_BATCH, _CAT_LEN))
    return np.concatenate(cols, axis=1)


_NW = 32
_ROWS_PER_W = _BATCH // _NW
_CHUNK = 128
_NVEC = _CAT_W // _LANES
_GDN = lax.GatherDimensionNumbers(
    offset_dims=(), collapsed_slice_dims=(0,), start_index_map=(0,))


def _shuf(v, idx):
    return lax.gather(v, idx[:, None], _GDN, (1,),
                      mode=lax.GatherScatterMode.PROMISE_IN_BOUNDS)


def _sc_body(x_hbm, g_hbm, o_hbm, xb, gb, ob):
    wid = lax.axis_index("s") * 2 + lax.axis_index("c")
    lanes = lax.iota(jnp.int32, _LANES)
    revpos = (_CAT_LEN - 1) - (lanes & (_CAT_LEN - 1))
    bfly = [lanes ^ s for s in (1, 2, 4)]
    idx_lo = jnp.maximum(lanes - _NUM_LEN, 0)
    idx_hi = (lanes + 3) & (_LANES - 1)
    is_num = lanes < _NUM_LEN

    for chunk in range(_ROWS_PER_W // _CHUNK):
        r0 = wid * _ROWS_PER_W + chunk * _CHUNK
        pltpu.sync_copy(x_hbm.at[pl.ds(r0, _CHUNK), :], xb)
        pltpu.sync_copy(g_hbm.at[pl.ds(r0, _CHUNK), :], gb)

        def row_body(r, carry):
            va = xb[r, pl.ds(0, _LANES)]
            a = jnp.abs(va) * np.float32(2.0)
            e = jnp.exp(a)
            th = np.float32(1.0) - np.float32(2.0) / (e + np.float32(1.0))
            th = jnp.where(va < np.float32(0.0), -th, th)
            ohs = []
            for m in range(_NVEC):
                xv = xb[r, pl.ds(_NUM_LEN + _LANES * m, _LANES)]
                gv = gb[r, pl.ds(_LANES * m, _LANES)]
                v = xv * np.float32(1.0 / _TAU) + gv
                b = lax.bitcast_convert_type(v, jnp.int32)
                t = b ^ ((b >> 31) & np.int32(0x7FFFFFFF))
                key = (t & np.int32(-8)) | revpos
                mx = key
                for ix in bfly:
                    mx = jnp.maximum(mx, _shuf(mx, ix))
                ohs.append(jnp.where(key == mx, np.float32(1.0), np.float32(0.0)))
            ooff = r * _WIDTH
            ob[pl.ds(ooff, _LANES)] = jnp.where(is_num, th, _shuf(ohs[0], idx_lo))
            for k in range(1, _NVEC + 1):
                lo = _shuf(ohs[k - 1], idx_hi)
                hi = _shuf(ohs[min(k, _NVEC - 1)], idx_lo)
                ob[pl.ds(ooff + _LANES * k, _LANES)] = jnp.where(is_num, lo, hi)
            return carry

        lax.fori_loop(0, _CHUNK, row_body, 0)
        pltpu.sync_copy(ob.at[pl.ds(0, _CHUNK * _WIDTH)],
                        o_hbm.at[pl.ds(r0 * _WIDTH, _CHUNK * _WIDTH)])


def kernel(x):
    g = jnp.asarray(_noise_table())
    mesh = plsc.VectorSubcoreMesh(core_axis_name="c", subcore_axis_name="s")
    f = pl.kernel(
        _sc_body,
        out_type=jax.ShapeDtypeStruct((_BATCH * _WIDTH,), jnp.float32),
        mesh=mesh,
        scratch_types=[
            pltpu.VMEM((_CHUNK, _WIDTH), jnp.float32),
            pltpu.VMEM((_CHUNK, _CAT_W), jnp.float32),
            pltpu.VMEM((_CHUNK * _WIDTH + _LANES,), jnp.float32),
        ],
    )
    return f(x, g).reshape(_BATCH, _WIDTH)


# speedup vs baseline: 1.0910x; 1.0910x over previous
import functools

import numpy as np
import jax
import jax.numpy as jnp
from jax import lax
from jax.experimental import pallas as pl
from jax.experimental.pallas import tpu as pltpu
from jax.experimental.pallas import tpu_sc as plsc

_NUM_LEN = 13
_N_CAT = 26
_CAT_LEN = 8
_TAU = 0.2
_BATCH = 16384
_TOL = 1e-20
_WIDTH = _NUM_LEN + _N_CAT * _CAT_LEN
_CAT_W = _N_CAT * _CAT_LEN
_LANES = 16


def _rotl(x, r):
    return ((x << np.uint32(r)) | (x >> np.uint32(32 - r))).astype(np.uint32)


def _threefry2x32(k0, k1, c0, c1):
    ks0 = np.uint32(k0)
    ks1 = np.uint32(k1)
    ks2 = np.uint32(ks0 ^ ks1 ^ np.uint32(0x1BD11BDA))
    ks = (ks0, ks1, ks2)
    rot = ((13, 15, 26, 6), (17, 29, 16, 24))
    x0 = (c0 + ks0).astype(np.uint32)
    x1 = (c1 + ks1).astype(np.uint32)
    for i in range(5):
        for r in rot[i % 2]:
            x0 = (x0 + x1).astype(np.uint32)
            x1 = _rotl(x1, r)
            x1 = (x1 ^ x0).astype(np.uint32)
        x0 = (x0 + ks[(i + 1) % 3]).astype(np.uint32)
        x1 = (x1 + ks[(i + 2) % 3] + np.uint32(i + 1)).astype(np.uint32)
    return x0, x1


def _fold_in(k0, k1, data):
    o0, o1 = _threefry2x32(k0, k1, np.zeros(1, np.uint32),
                           np.full(1, data, np.uint32))
    return int(o0[0]), int(o1[0])


def _random_bits(k0, k1, n):
    c0 = np.zeros(n, dtype=np.uint32)
    c1 = np.arange(n, dtype=np.uint32)
    o0, o1 = _threefry2x32(k0, k1, c0, c1)
    return (o0 ^ o1).astype(np.uint32)


def _bits_to_unit_float(bits):
    fb = (bits >> np.uint32(9)) | np.uint32(0x3F800000)
    return fb.view(np.float32) - np.float32(1.0)


@functools.cache
def _noise_table():
    n = _BATCH * _CAT_LEN
    tiny = np.float64(np.finfo(np.float32).tiny)
    cols = []
    for i in range(_N_CAT):
        ka = _fold_in(0, 0, i)
        kb = _fold_in(0, 0, 10000 + i)
        u1 = _bits_to_unit_float(_random_bits(ka[0], ka[1], n)).astype(np.float64)
        g1 = -np.log(-np.log(u1 + _TOL) + _TOL)
        f2 = _bits_to_unit_float(_random_bits(kb[0], kb[1], n)).astype(np.float64)
        u2 = np.maximum(tiny, f2 * (1.0 - tiny) + tiny)
        g2 = -np.log(-np.log(u2))
        cols.append((g1 / _TAU + g2).astype(np.float32).reshape(_BATCH, _CAT_LEN))
    return np.concatenate(cols, axis=1)


_NW = 32
_ROWS_PER_W = _BATCH // _NW
_CHUNK = 128
_NVEC = _CAT_W // _LANES
_GDN = lax.GatherDimensionNumbers(
    offset_dims=(), collapsed_slice_dims=(0,), start_index_map=(0,))


def _shuf(v, idx):
    return lax.gather(v, idx[:, None], _GDN, (1,),
                      mode=lax.GatherScatterMode.PROMISE_IN_BOUNDS)


def _sc_body(x_hbm, g_hbm, o_hbm, xb, gb, ob, semx, semg, semo):
    wid = lax.axis_index("s") * 2 + lax.axis_index("c")
    lanes = lax.iota(jnp.int32, _LANES)
    revpos = (_CAT_LEN - 1) - (lanes & (_CAT_LEN - 1))
    bfly = [lanes ^ s for s in (1, 2, 4)]
    idx_lo = jnp.maximum(lanes - _NUM_LEN, 0)
    idx_hi = (lanes + 3) & (_LANES - 1)
    is_num = lanes < _NUM_LEN

    prev_out = None
    for chunk in range(_ROWS_PER_W // _CHUNK):
        r0 = wid * _ROWS_PER_W + chunk * _CHUNK
        cx = pltpu.async_copy(
            x_hbm.at[pl.ds(r0 * _WIDTH, _CHUNK * _WIDTH)], xb, semx)
        cg = pltpu.async_copy(
            g_hbm.at[pl.ds(r0 * _CAT_W, _CHUNK * _CAT_W)], gb, semg)
        if prev_out is not None:
            prev_out.wait()
        cx.wait()
        cg.wait()

        def row_body(r, carry):
            xoff = r * _WIDTH
            goff = r * _CAT_W
            va = xb[pl.ds(xoff, _LANES)]
            a = jnp.abs(va) * np.float32(2.0)
            e = jnp.exp(a)
            th = np.float32(1.0) - np.float32(2.0) / (e + np.float32(1.0))
            th = jnp.where(va < np.float32(0.0), -th, th)
            ohs = []
            for m in range(_NVEC):
                xv = xb[pl.ds(xoff + _NUM_LEN + _LANES * m, _LANES)]
                gv = gb[pl.ds(goff + _LANES * m, _LANES)]
                v = xv * np.float32(1.0 / _TAU) + gv
                b = lax.bitcast_convert_type(v, jnp.int32)
                t = b ^ ((b >> 31) & np.int32(0x7FFFFFFF))
                key = (t & np.int32(-8)) | revpos
                mx = key
                for ix in bfly:
                    mx = jnp.maximum(mx, _shuf(mx, ix))
                ohs.append(jnp.where(key == mx, np.float32(1.0), np.float32(0.0)))
            ooff = r * _WIDTH
            ob[pl.ds(ooff, _LANES)] = jnp.where(is_num, th, _shuf(ohs[0], idx_lo))
            for k in range(1, _NVEC + 1):
                lo = _shuf(ohs[k - 1], idx_hi)
                hi = _shuf(ohs[min(k, _NVEC - 1)], idx_lo)
                ob[pl.ds(ooff + _LANES * k, _LANES)] = jnp.where(is_num, lo, hi)
            return carry

        lax.fori_loop(0, _CHUNK, row_body, 0)
        prev_out = pltpu.async_copy(
            ob.at[pl.ds(0, _CHUNK * _WIDTH)],
            o_hbm.at[pl.ds(r0 * _WIDTH, _CHUNK * _WIDTH)], semo)
    prev_out.wait()


def kernel(x):
    g = jnp.asarray(_noise_table())
    mesh = plsc.VectorSubcoreMesh(core_axis_name="c", subcore_axis_name="s")
    f = pl.kernel(
        _sc_body,
        out_type=jax.ShapeDtypeStruct((_BATCH * _WIDTH,), jnp.float32),
        mesh=mesh,
        scratch_types=[
            pltpu.VMEM((_CHUNK * _WIDTH,), jnp.float32),
            pltpu.VMEM((_CHUNK * _CAT_W,), jnp.float32),
            pltpu.VMEM((_CHUNK * _WIDTH + _LANES,), jnp.float32),
            pltpu.SemaphoreType.DMA,
            pltpu.SemaphoreType.DMA,
            pltpu.SemaphoreType.DMA,
        ],
    )
    return f(x.reshape(-1), g.reshape(-1)).reshape(_BATCH, _WIDTH)


# speedup vs baseline: 1.0911x; 1.0001x over previous
import functools

import numpy as np
import jax
import jax.numpy as jnp
from jax import lax
from jax.experimental import pallas as pl
from jax.experimental.pallas import tpu as pltpu
from jax.experimental.pallas import tpu_sc as plsc

_NUM_LEN = 13
_N_CAT = 26
_CAT_LEN = 8
_TAU = 0.2
_BATCH = 16384
_TOL = 1e-20
_WIDTH = _NUM_LEN + _N_CAT * _CAT_LEN
_CAT_W = _N_CAT * _CAT_LEN
_LANES = 16


def _rotl(x, r):
    return ((x << np.uint32(r)) | (x >> np.uint32(32 - r))).astype(np.uint32)


def _threefry2x32(k0, k1, c0, c1):
    ks0 = np.uint32(k0)
    ks1 = np.uint32(k1)
    ks2 = np.uint32(ks0 ^ ks1 ^ np.uint32(0x1BD11BDA))
    ks = (ks0, ks1, ks2)
    rot = ((13, 15, 26, 6), (17, 29, 16, 24))
    x0 = (c0 + ks0).astype(np.uint32)
    x1 = (c1 + ks1).astype(np.uint32)
    for i in range(5):
        for r in rot[i % 2]:
            x0 = (x0 + x1).astype(np.uint32)
            x1 = _rotl(x1, r)
            x1 = (x1 ^ x0).astype(np.uint32)
        x0 = (x0 + ks[(i + 1) % 3]).astype(np.uint32)
        x1 = (x1 + ks[(i + 2) % 3] + np.uint32(i + 1)).astype(np.uint32)
    return x0, x1


def _fold_in(k0, k1, data):
    o0, o1 = _threefry2x32(k0, k1, np.zeros(1, np.uint32),
                           np.full(1, data, np.uint32))
    return int(o0[0]), int(o1[0])


def _random_bits(k0, k1, n):
    c0 = np.zeros(n, dtype=np.uint32)
    c1 = np.arange(n, dtype=np.uint32)
    o0, o1 = _threefry2x32(k0, k1, c0, c1)
    return (o0 ^ o1).astype(np.uint32)


def _bits_to_unit_float(bits):
    fb = (bits >> np.uint32(9)) | np.uint32(0x3F800000)
    return fb.view(np.float32) - np.float32(1.0)


@functools.cache
def _noise_table():
    n = _BATCH * _CAT_LEN
    tiny = np.float64(np.finfo(np.float32).tiny)
    cols = []
    for i in range(_N_CAT):
        ka = _fold_in(0, 0, i)
        kb = _fold_in(0, 0, 10000 + i)
        u1 = _bits_to_unit_float(_random_bits(ka[0], ka[1], n)).astype(np.float64)
        g1 = -np.log(-np.log(u1 + _TOL) + _TOL)
        f2 = _bits_to_unit_float(_random_bits(kb[0], kb[1], n)).astype(np.float64)
        u2 = np.maximum(tiny, f2 * (1.0 - tiny) + tiny)
        g2 = -np.log(-np.log(u2))
        cols.append((g1 / _TAU + g2).astype(np.float32).reshape(_BATCH, _CAT_LEN))
    return np.ascontiguousarray(np.concatenate(cols, axis=1)).reshape(-1)


_NW = 32
_ROWS_PER_W = _BATCH // _NW
_CHUNK = 128
_NVEC = _CAT_W // _LANES
_GDN = lax.GatherDimensionNumbers(
    offset_dims=(), collapsed_slice_dims=(0,), start_index_map=(0,))


def _shuf(v, idx):
    return lax.gather(v, idx[:, None], _GDN, (1,),
                      mode=lax.GatherScatterMode.PROMISE_IN_BOUNDS)


def _sc_body(x_hbm, g_hbm, o_hbm, xb, gb, ob, semx, semg, semo):
    wid = lax.axis_index("s") * 2 + lax.axis_index("c")
    lanes = lax.iota(jnp.int32, _LANES)
    revpos = (_CAT_LEN - 1) - (lanes & (_CAT_LEN - 1))
    bfly = [lanes ^ s for s in (1, 2, 4)]
    idx_lo = jnp.maximum(lanes - _NUM_LEN, 0)
    idx_hi = (lanes + 3) & (_LANES - 1)
    is_num = lanes < _NUM_LEN

    prev_out = None
    for chunk in range(_ROWS_PER_W // _CHUNK):
        r0 = wid * _ROWS_PER_W + chunk * _CHUNK
        cx = pltpu.async_copy(
            x_hbm.at[pl.ds(r0 * _WIDTH, _CHUNK * _WIDTH)], xb, semx)
        cg = pltpu.async_copy(
            g_hbm.at[pl.ds(r0 * _CAT_W, _CHUNK * _CAT_W)], gb, semg)
        if prev_out is not None:
            prev_out.wait()
        cx.wait()
        cg.wait()

        def row_body(r, carry):
            xoff = r * _WIDTH
            goff = r * _CAT_W
            va = xb[pl.ds(xoff, _LANES)]
            a = jnp.abs(va) * np.float32(2.0)
            e = jnp.exp(a)
            th = np.float32(1.0) - np.float32(2.0) / (e + np.float32(1.0))
            th = jnp.where(va < np.float32(0.0), -th, th)
            ohs = []
            for m in range(_NVEC):
                xv = xb[pl.ds(xoff + _NUM_LEN + _LANES * m, _LANES)]
                gv = gb[pl.ds(goff + _LANES * m, _LANES)]
                v = xv * np.float32(1.0 / _TAU) + gv
                b = lax.bitcast_convert_type(v, jnp.int32)
                t = b ^ ((b >> 31) & np.int32(0x7FFFFFFF))
                key = (t & np.int32(-8)) | revpos
                mx = key
                for ix in bfly:
                    mx = jnp.maximum(mx, _shuf(mx, ix))
                ohs.append(jnp.where(key == mx, np.float32(1.0), np.float32(0.0)))
            ooff = r * _WIDTH
            ob[pl.ds(ooff, _LANES)] = jnp.where(is_num, th, _shuf(ohs[0], idx_lo))
            for k in range(1, _NVEC + 1):
                lo = _shuf(ohs[k - 1], idx_hi)
                hi = _shuf(ohs[min(k, _NVEC - 1)], idx_lo)
                ob[pl.ds(ooff + _LANES * k, _LANES)] = jnp.where(is_num, lo, hi)
            return carry

        lax.fori_loop(0, _CHUNK, row_body, 0)
        prev_out = pltpu.async_copy(
            ob.at[pl.ds(0, _CHUNK * _WIDTH)],
            o_hbm.at[pl.ds(r0 * _WIDTH, _CHUNK * _WIDTH)], semo)
    prev_out.wait()


def kernel(x):
    g = jnp.asarray(_noise_table())
    mesh = plsc.VectorSubcoreMesh(core_axis_name="c", subcore_axis_name="s")
    f = pl.kernel(
        _sc_body,
        out_type=jax.ShapeDtypeStruct((_BATCH * _WIDTH,), jnp.float32),
        mesh=mesh,
        scratch_types=[
            pltpu.VMEM((_CHUNK * _WIDTH,), jnp.float32),
            pltpu.VMEM((_CHUNK * _CAT_W,), jnp.float32),
            pltpu.VMEM((_CHUNK * _WIDTH + _LANES,), jnp.float32),
            pltpu.SemaphoreType.DMA,
            pltpu.SemaphoreType.DMA,
            pltpu.SemaphoreType.DMA,
        ],
    )
    return f(x.reshape(-1), g).reshape(_BATCH, _WIDTH)


# speedup vs baseline: 1.0913x; 1.0002x over previous
import functools

import numpy as np
import jax
import jax.numpy as jnp
from jax import lax
from jax.experimental import pallas as pl
from jax.experimental.pallas import tpu as pltpu
from jax.experimental.pallas import tpu_sc as plsc

_NUM_LEN = 13
_N_CAT = 26
_CAT_LEN = 8
_TAU = 0.2
_BATCH = 16384
_TOL = 1e-20
_WIDTH = _NUM_LEN + _N_CAT * _CAT_LEN
_CAT_W = _N_CAT * _CAT_LEN
_LANES = 16


def _rotl(x, r):
    return ((x << np.uint32(r)) | (x >> np.uint32(32 - r))).astype(np.uint32)


def _threefry2x32(k0, k1, c0, c1):
    ks0 = np.uint32(k0)
    ks1 = np.uint32(k1)
    ks2 = np.uint32(ks0 ^ ks1 ^ np.uint32(0x1BD11BDA))
    ks = (ks0, ks1, ks2)
    rot = ((13, 15, 26, 6), (17, 29, 16, 24))
    x0 = (c0 + ks0).astype(np.uint32)
    x1 = (c1 + ks1).astype(np.uint32)
    for i in range(5):
        for r in rot[i % 2]:
            x0 = (x0 + x1).astype(np.uint32)
            x1 = _rotl(x1, r)
            x1 = (x1 ^ x0).astype(np.uint32)
        x0 = (x0 + ks[(i + 1) % 3]).astype(np.uint32)
        x1 = (x1 + ks[(i + 2) % 3] + np.uint32(i + 1)).astype(np.uint32)
    return x0, x1


def _fold_in(k0, k1, data):
    o0, o1 = _threefry2x32(k0, k1, np.zeros(1, np.uint32),
                           np.full(1, data, np.uint32))
    return int(o0[0]), int(o1[0])


def _random_bits(k0, k1, n):
    c0 = np.zeros(n, dtype=np.uint32)
    c1 = np.arange(n, dtype=np.uint32)
    o0, o1 = _threefry2x32(k0, k1, c0, c1)
    return (o0 ^ o1).astype(np.uint32)


def _bits_to_unit_float(bits):
    fb = (bits >> np.uint32(9)) | np.uint32(0x3F800000)
    return fb.view(np.float32) - np.float32(1.0)


@functools.cache
def _noise_table():
    n = _BATCH * _CAT_LEN
    tiny = np.float64(np.finfo(np.float32).tiny)
    cols = []
    for i in range(_N_CAT):
        ka = _fold_in(0, 0, i)
        kb = _fold_in(0, 0, 10000 + i)
        u1 = _bits_to_unit_float(_random_bits(ka[0], ka[1], n)).astype(np.float64)
        g1 = -np.log(-np.log(u1 + _TOL) + _TOL)
        f2 = _bits_to_unit_float(_random_bits(kb[0], kb[1], n)).astype(np.float64)
        u2 = np.maximum(tiny, f2 * (1.0 - tiny) + tiny)
        g2 = -np.log(-np.log(u2))
        cols.append((g1 / _TAU + g2).astype(np.float32).reshape(_BATCH, _CAT_LEN))
    return np.ascontiguousarray(np.concatenate(cols, axis=1)).reshape(-1)


_NW = 32
_ROWS_PER_W = _BATCH // _NW
_CHUNK = 128
_NVEC = _CAT_W // _LANES
_GDN = lax.GatherDimensionNumbers(
    offset_dims=(), collapsed_slice_dims=(0,), start_index_map=(0,))


def _shuf(v, idx):
    return lax.gather(v, idx[:, None], _GDN, (1,),
                      mode=lax.GatherScatterMode.PROMISE_IN_BOUNDS)


def _sc_body(x_hbm, g_hbm, o_hbm, xb, gb, ob, semx, semg, semo):
    wid = lax.axis_index("s") * 2 + lax.axis_index("c")
    lanes = lax.iota(jnp.int32, _LANES)
    revpos = (_CAT_LEN - 1) - (lanes & (_CAT_LEN - 1))
    bfly = [lanes ^ s for s in (1, 2, 4)]
    idx_lo = jnp.maximum(lanes - _NUM_LEN, 0)
    idx_hi = (lanes + 3) & (_LANES - 1)
    is_num = lanes < _NUM_LEN

    prev_out = None
    for chunk in range(_ROWS_PER_W // _CHUNK):
        r0 = wid * _ROWS_PER_W + chunk * _CHUNK
        cx = pltpu.async_copy(
            x_hbm.at[pl.ds(r0 * _WIDTH, _CHUNK * _WIDTH)], xb, semx)
        cg = pltpu.async_copy(
            g_hbm.at[pl.ds(r0 * _CAT_W, _CHUNK * _CAT_W)], gb, semg)
        if prev_out is not None:
            prev_out.wait()
        cx.wait()
        cg.wait()

        def process_row(r):
            xoff = r * _WIDTH
            goff = r * _CAT_W
            va = xb[pl.ds(xoff, _LANES)]
            a = jnp.abs(va) * np.float32(2.0)
            e = jnp.exp(a)
            th = np.float32(1.0) - np.float32(2.0) / (e + np.float32(1.0))
            th = jnp.where(va < np.float32(0.0), -th, th)
            ohs = []
            for m in range(_NVEC):
                xv = xb[pl.ds(xoff + _NUM_LEN + _LANES * m, _LANES)]
                gv = gb[pl.ds(goff + _LANES * m, _LANES)]
                v = xv * np.float32(1.0 / _TAU) + gv
                b = lax.bitcast_convert_type(v, jnp.int32)
                t = b ^ ((b >> 31) & np.int32(0x7FFFFFFF))
                key = (t & np.int32(-8)) | revpos
                mx = key
                for ix in bfly:
                    mx = jnp.maximum(mx, _shuf(mx, ix))
                ohs.append(jnp.where(key == mx, np.float32(1.0), np.float32(0.0)))
            ooff = r * _WIDTH
            ob[pl.ds(ooff, _LANES)] = jnp.where(is_num, th, _shuf(ohs[0], idx_lo))
            for k in range(1, _NVEC + 1):
                lo = _shuf(ohs[k - 1], idx_hi)
                hi = _shuf(ohs[min(k, _NVEC - 1)], idx_lo)
                ob[pl.ds(ooff + _LANES * k, _LANES)] = jnp.where(is_num, lo, hi)

        def row_body(r, carry):
            process_row(r * 2)
            process_row(r * 2 + 1)
            return carry

        lax.fori_loop(0, _CHUNK // 2, row_body, 0)
        prev_out = pltpu.async_copy(
            ob.at[pl.ds(0, _CHUNK * _WIDTH)],
            o_hbm.at[pl.ds(r0 * _WIDTH, _CHUNK * _WIDTH)], semo)
    prev_out.wait()


def kernel(x):
    g = jnp.asarray(_noise_table())
    mesh = plsc.VectorSubcoreMesh(core_axis_name="c", subcore_axis_name="s")
    f = pl.kernel(
        _sc_body,
        out_type=jax.ShapeDtypeStruct((_BATCH * _WIDTH,), jnp.float32),
        mesh=mesh,
        scratch_types=[
            pltpu.VMEM((_CHUNK * _WIDTH,), jnp.float32),
            pltpu.VMEM((_CHUNK * _CAT_W,), jnp.float32),
            pltpu.VMEM((_CHUNK * _WIDTH + _LANES,), jnp.float32),
            pltpu.SemaphoreType.DMA,
            pltpu.SemaphoreType.DMA,
            pltpu.SemaphoreType.DMA,
        ],
    )
    return f(x.reshape(-1), g).reshape(_BATCH, _WIDTH)
